# baseline (speedup 1.0000x reference)
import functools
import math

import jax
import jax.numpy as jnp
from jax import lax
from jax.experimental import pallas as pl
from jax.experimental.pallas import tpu as pltpu
from jax.experimental.pallas import tpu_sc as plsc

VOCAB = 1000000
EMBED = 64
EPAD = 128
LANES = 16
NUM_CORES = 2
NUM_SUBCORES = 16
NUM_WORKERS = NUM_CORES * NUM_SUBCORES
NB = 4
SCALE = math.sqrt(EMBED)


def _emb_lookup(table_p, idx4, b, l):
    nw, n_blocks, two, half = idx4.shape
    b_per_w = b // NUM_WORKERS

    mesh = plsc.VectorSubcoreMesh(core_axis_name="c", subcore_axis_name="s")

    @functools.partial(
        pl.kernel,
        mesh=mesh,
        out_type=jax.ShapeDtypeStruct((b, l, EMBED), jnp.float32),
        scratch_types=[
            pltpu.VMEM((n_blocks, two, half), jnp.int32),
            pltpu.VMEM((NB * l, EPAD), jnp.float32),
            pltpu.VMEM((NB, l, EMBED), jnp.float32),
            pltpu.SemaphoreType.DMA,
        ],
        compiler_params=pltpu.CompilerParams(use_tc_tiling_on_sc=True),
    )
    def k(table_hbm, idx_hbm, out_hbm, idx_v, gbuf, sbuf, sem):
        wid = lax.axis_index("s") * NUM_CORES + lax.axis_index("c")
        b0w = wid * b_per_w
        pltpu.sync_copy(idx_hbm.at[wid], idx_v)

        def block_body(j, carry):
            pltpu.async_copy(
                table_hbm.at[idx_v.at[j, 0]], gbuf.at[pl.ds(0, half)], sem
            )
            pltpu.async_copy(
                table_hbm.at[idx_v.at[j, 1]], gbuf.at[pl.ds(half, half)], sem
            )
            pltpu.make_async_copy(
                table_hbm.at[idx_v.at[j, 0]], gbuf.at[pl.ds(0, half)], sem
            ).wait()
            pltpu.make_async_copy(
                table_hbm.at[idx_v.at[j, 1]], gbuf.at[pl.ds(half, half)], sem
            ).wait()

            for ib in range(NB):

                def row_body(il, c2, _ib=ib):
                    for q in range(EMBED // LANES):
                        sl = pl.ds(q * LANES, LANES)
                        sbuf[_ib, il, sl] = gbuf[_ib * l + il, sl] * SCALE
                    return c2

                lax.fori_loop(0, l, row_body, 0, unroll=5)

            pltpu.sync_copy(sbuf, out_hbm.at[pl.ds(b0w + j * NB, NB)])
            return carry

        lax.fori_loop(0, n_blocks, block_body, 0)

    return k(table_p, idx4)


def kernel(input, table):
    b, l = input.shape
    b_per_w = b // NUM_WORKERS
    n_blocks = b_per_w // NB
    idx4 = input.reshape(NUM_WORKERS, n_blocks, 2, (NB // 2) * l).astype(jnp.int32)
    table_p = jnp.pad(table, ((0, 0), (0, EPAD - EMBED)))
    return _emb_lookup(table_p, idx4, b, l)
